# baseline (speedup 1.0000x reference)
import jax
import jax.numpy as jnp
from jax import lax
from jax.experimental import pallas as pl
from jax.experimental.pallas import tpu as pltpu
from jax.experimental.pallas import tpu_sc as plsc

B = 16384
L = 20
D = 64
L2 = 2 * L
D2 = D // 2
NH = D2 // 16
NC = 2
NS = 16
LANES = 16
NW = NC * NS
PER_W = B // NW
C = 16
NCH = PER_W // C
IDS = C * L2
WIDS = PER_W * L2


def _body(ix_hbm, iy_hbm, tab_hbm, out_hbm,
          ixa, iya, rxv0, rxv1, ryv0, ryv1,
          stage, oacc, sem0, sem1):
    wid = lax.axis_index("s") * NC + lax.axis_index("c")
    lane = lax.iota(jnp.int32, LANES)
    rxv = (rxv0, rxv1)
    ryv = (ryv0, ryv1)
    sems = (sem0, sem1)

    pltpu.sync_copy(ix_hbm.at[pl.ds(wid * WIDS, WIDS)], ixa)
    pltpu.sync_copy(iy_hbm.at[pl.ds(wid * WIDS, WIDS)], iya)

    def fire(ch, b):
        pltpu.async_copy(tab_hbm.at[ixa.at[pl.ds(ch * IDS, IDS)]], rxv[b], sems[b])
        pltpu.async_copy(tab_hbm.at[iya.at[pl.ds(ch * IDS, IDS)]], ryv[b], sems[b])

    def drain(b):
        pltpu.make_async_copy(tab_hbm.at[pl.ds(0, IDS), :], rxv[b], sems[b]).wait()
        pltpu.make_async_copy(tab_hbm.at[pl.ds(0, IDS), :], ryv[b], sems[b]).wait()

    def step(ch, b):
        nxt = ch + 1

        @pl.when(nxt < NCH)
        def _():
            fire(nxt, 1 - b)

        drain(b)
        rx, ry = rxv[b], ryv[b]

        def row(r, rcarry):
            base = r * L2
            axe = [rx[base, pl.ds(h * LANES, LANES)] for h in range(NH)]
            axo = [rx[base + 1, pl.ds(h * LANES, LANES)] for h in range(NH)]
            aye = [ry[base, pl.ds(h * LANES, LANES)] for h in range(NH)]
            ayo = [ry[base + 1, pl.ds(h * LANES, LANES)] for h in range(NH)]
            for l in range(1, L):
                for h in range(NH):
                    axe[h] = axe[h] + rx[base + 2 * l, pl.ds(h * LANES, LANES)]
                    axo[h] = axo[h] + rx[base + 2 * l + 1, pl.ds(h * LANES, LANES)]
                    aye[h] = aye[h] + ry[base + 2 * l, pl.ds(h * LANES, LANES)]
                    ayo[h] = ayo[h] + ry[base + 2 * l + 1, pl.ds(h * LANES, LANES)]
            dot = (axe[0] * aye[0] + axo[0] * ayo[0]) + \
                  (axe[1] * aye[1] + axo[1] * ayo[1])
            stage[r, :] = dot
            return rcarry

        lax.fori_loop(0, C, row, 0)

        dot = plsc.load_gather(stage, [lane, lax.broadcast(0, (LANES,))])
        for j in range(1, LANES):
            dot = dot + plsc.load_gather(stage, [lane, lax.broadcast(j, (LANES,))])
        y = 1.0 / (1.0 + jnp.exp(-dot))
        oacc[pl.ds(ch * C, C)] = y

    fire(0, 0)

    def pair(i, carry):
        step(2 * i, 0)
        step(2 * i + 1, 1)
        return carry

    lax.fori_loop(0, NCH // 2, pair, 0)
    pltpu.sync_copy(oacc, out_hbm.at[pl.ds(wid * PER_W, PER_W)])


def kernel(idx_x, idx_y, table):
    v = table.shape[0]

    def dbl(idx):
        i2 = idx.astype(jnp.int32) * 2
        return jnp.stack([i2, i2 + 1], axis=-1).reshape(-1)

    ix = dbl(idx_x)
    iy = dbl(idx_y)
    tb = table.reshape(2 * v, D2)
    mesh = plsc.VectorSubcoreMesh(core_axis_name="c", subcore_axis_name="s")
    f = pl.kernel(
        _body,
        out_type=jax.ShapeDtypeStruct((B,), jnp.float32),
        mesh=mesh,
        compiler_params=pltpu.CompilerParams(
            needs_layout_passes=False, use_tc_tiling_on_sc=False),
        scratch_types=[
            pltpu.VMEM((WIDS,), jnp.int32),
            pltpu.VMEM((WIDS,), jnp.int32),
            pltpu.VMEM((IDS, D2), jnp.float32),
            pltpu.VMEM((IDS, D2), jnp.float32),
            pltpu.VMEM((IDS, D2), jnp.float32),
            pltpu.VMEM((IDS, D2), jnp.float32),
            pltpu.VMEM((C, LANES), jnp.float32),
            pltpu.VMEM((PER_W,), jnp.float32),
            pltpu.SemaphoreType.DMA,
            pltpu.SemaphoreType.DMA,
        ],
    )
    return f(ix, iy, tb)
